# baseline (speedup 1.0000x reference)
import functools

import numpy as np

import jax
import jax.numpy as jnp
from jax import lax
from jax.experimental import pallas as pl
from jax.experimental.pallas import tpu as pltpu
from jax.experimental.pallas import tpu_sc as plsc

_BM = 1024
_K = 8
_RES = 256
_PLANE = _RES * _RES
_CHUNK = 128
_NW = 32

_INTERPRET = False


def _knn_body(coord_ref, pts_ref, dist_ref, idx_ref, s_ref, *,
              P, n0):
    n = n0 + pl.program_id(0)
    mb = pl.program_id(1)
    c = jnp.clip(coord_ref[0], -0.999, 0.999)
    pt = pts_ref[0]
    d2 = jnp.zeros((_BM, P), jnp.float32)
    for a in range(3):
        diff = c[:, a:a + 1] - pt[a:a + 1, :]
        d2 = d2 + diff * diff
    colidx_f = lax.broadcasted_iota(jnp.int32, (_BM, P), 1
                                    ).astype(jnp.float32)
    dcols = []
    icols = []
    for _ in range(_K):
        m = jnp.min(d2, axis=1, keepdims=True)
        am_f = jnp.min(jnp.where(d2 == m, colidx_f, jnp.float32(P)),
                       axis=1, keepdims=True)
        dcols.append(m)
        icols.append(am_f.astype(jnp.int32))
        d2 = jnp.where(colidx_f == am_f, jnp.float32(jnp.inf), d2)
    dist = jnp.concatenate(dcols, axis=1)
    dist_ref[0] = dist
    idx_ref[0] = jnp.concatenate(icols, axis=1) + n * P

    part = jnp.sum(1.0 / dist, axis=0, keepdims=True)

    @pl.when(mb == 0)
    def _():
        s_ref[0] = part

    @pl.when(mb != 0)
    def _():
        s_ref[0] = s_ref[0] + part


def _texidx_body(coord_ref, tex_ref, *, n0):
    n = n0 + pl.program_id(0)
    c = jnp.clip(coord_ref[0], -0.999, 0.999)
    cols = []
    for p, (a, b) in enumerate(((0, 1), (0, 2), (1, 2))):
        u = c[:, a:a + 1]
        v = c[:, b:b + 1]
        x = (u + 1.0) * 0.5 * float(_RES) - 0.5
        y = (v + 1.0) * 0.5 * float(_RES) - 0.5
        x0i = jnp.clip(jnp.floor(x).astype(jnp.int32), 0, _RES - 1)
        y0i = jnp.clip(jnp.floor(y).astype(jnp.int32), 0, _RES - 1)
        x1i = jnp.minimum(x0i + 1, _RES - 1)
        y1i = jnp.minimum(y0i + 1, _RES - 1)
        base = (n * 3 + p) * _PLANE
        cols += [base + y0i * _RES + x0i, base + y0i * _RES + x1i,
                 base + y1i * _RES + x0i, base + y1i * _RES + x1i]
    tex_ref[0] = jnp.concatenate(cols, axis=1)


def _texidx_call(coordinates, n0):
    N, M, _ = coordinates.shape
    grid = (N, M // _BM)
    return pl.pallas_call(
        functools.partial(_texidx_body, n0=n0),
        grid=grid,
        in_specs=[pl.BlockSpec((1, _BM, 3), lambda n, mb: (n, mb, 0))],
        out_specs=[pl.BlockSpec((1, _BM, 12), lambda n, mb: (n, mb, 0))],
        out_shape=[jax.ShapeDtypeStruct((N, M, 12), jnp.int32)],
        interpret=_INTERPRET,
    )(coordinates)


def _knn_call(coordinates, pts_t, n0):
    N, M, _ = coordinates.shape
    P = pts_t.shape[2]
    grid = (N, M // _BM)
    return pl.pallas_call(
        functools.partial(_knn_body, P=P, n0=n0),
        grid=grid,
        in_specs=[
            pl.BlockSpec((1, _BM, 3), lambda n, mb: (n, mb, 0)),
            pl.BlockSpec((1, 3, P), lambda n, mb: (n, 0, 0)),
        ],
        out_specs=[
            pl.BlockSpec((1, _BM, _K), lambda n, mb: (n, mb, 0)),
            pl.BlockSpec((1, _BM, _K), lambda n, mb: (n, mb, 0)),
            pl.BlockSpec((1, 1, _K), lambda n, mb: (n, 0, 0)),
        ],
        out_shape=[
            jax.ShapeDtypeStruct((N, M, _K), jnp.float32),
            jax.ShapeDtypeStruct((N, M, _K), jnp.int32),
            jax.ShapeDtypeStruct((N, 1, _K), jnp.float32),
        ],
        interpret=_INTERPRET,
    )(coordinates, pts_t)


def _sc_gather(table, idx3):
    C = idx3.shape[1]
    D = table.shape[1]
    B = _NW * C * _CHUNK
    mesh = plsc.VectorSubcoreMesh(core_axis_name="c", subcore_axis_name="s")

    @functools.partial(
        pl.kernel,
        mesh=mesh,
        out_type=jax.ShapeDtypeStruct((B, D), jnp.float32),
        scratch_types=[
            pltpu.VMEM((C, _CHUNK), jnp.int32),
            pltpu.VMEM((C, _CHUNK, D), jnp.float32),
            pltpu.SemaphoreType.DMA((C,)),
            pltpu.SemaphoreType.DMA((C,)),
        ],
        compiler_params=pltpu.CompilerParams(use_tc_tiling_on_sc=False),
    )
    def gk(tab_hbm, idx_hbm, out_hbm, idx_v, rows_v, gsem, wsem):
        wid = lax.axis_index("s") * 2 + lax.axis_index("c")
        pltpu.sync_copy(idx_hbm.at[wid], idx_v)
        gcp = [pltpu.async_copy(tab_hbm.at[idx_v.at[j]], rows_v.at[j],
                                gsem.at[j]) for j in range(C)]
        wcp = []
        for j in range(C):
            gcp[j].wait()
            wcp.append(pltpu.async_copy(
                rows_v.at[j],
                out_hbm.at[pl.ds(wid * (C * _CHUNK) + j * _CHUNK, _CHUNK)],
                wsem.at[j]))
        for j in range(C):
            wcp[j].wait()

    return gk(table, idx3)


def _contract0(a, b):
    return lax.dot_general(a, b, (((0,), (0,)), ((), ())),
                           preferred_element_type=jnp.float32)


def _decode_body(coord_ref, coord_t_ref, dir_t_ref, dist_ref, s_ref,
                 comb_ref, tex_ref,
                 pqw0_ref, pqb0_ref, pqw1_ref, pqb1_ref,
                 fw0_ref, fb0_ref, fw1_ref, fb1_ref, fw2_ref, fb2_ref,
                 dw_ref, db_ref, rw0a_ref, rw0b_ref, rb0_ref,
                 rw1_ref, rb1_ref,
                 dens_ref, rgb_ref):
    c = jnp.clip(coord_ref[0], -0.999, 0.999)
    c_t = jnp.clip(coord_t_ref[0], -0.999, 0.999)

    tex_rows = tex_ref[0]
    tex = jnp.zeros((_BM, 32), jnp.float32)
    for p, (a, b) in enumerate(((0, 1), (0, 2), (1, 2))):
        u = c[:, a:a + 1]
        v = c[:, b:b + 1]
        x = (u + 1.0) * 0.5 * float(_RES) - 0.5
        y = (v + 1.0) * 0.5 * float(_RES) - 0.5
        wx = x - jnp.floor(x)
        wy = y - jnp.floor(y)
        w4 = ((1 - wx) * (1 - wy), wx * (1 - wy), (1 - wx) * wy, wx * wy)
        for ci in range(4):
            j = p * 4 + ci
            tex = tex + w4[ci] * tex_rows[:, j * 32:(j + 1) * 32]
    tex = tex / 3.0

    comb = comb_ref[0]
    comb_t = comb.T
    pf_t = jnp.concatenate(
        [comb_t[k * 48:k * 48 + 32, :] for k in range(_K)], axis=1)
    pos_t = jnp.concatenate(
        [comb_t[k * 48 + 32:k * 48 + 35, :] for k in range(_K)], axis=1)
    rel_t = jnp.concatenate([c_t] * _K, axis=1) - pos_t
    sins = [jnp.sin(rel_t)]
    coss = [jnp.cos(rel_t)]
    for _ in range(5):
        s_p, c_p = sins[-1], coss[-1]
        sins.append(2.0 * s_p * c_p)
        coss.append(1.0 - 2.0 * s_p * s_p)
    x_t = jnp.concatenate([pf_t] + sins + coss + [rel_t],
                          axis=0)

    wcols = []
    s = s_ref[0]
    dist = dist_ref[0]
    for k in range(_K):
        wcols.append((1.0 / dist[:, k:k + 1]) / s[0:1, k:k + 1])
    h = jnp.maximum(_contract0(x_t, pqw0_ref[...]) + pqb0_ref[...], 0.0)
    pfo = jnp.dot(h, pqw1_ref[...],
                  preferred_element_type=jnp.float32) + pqb1_ref[...]
    spf = jnp.zeros((_BM, 32), jnp.float32)
    for k in range(_K):
        spf = spf + pfo[k * _BM:(k + 1) * _BM] * wcols[k]

    feat = jnp.concatenate([tex, spf], axis=1)
    feat = jnp.maximum(
        jnp.dot(feat, fw0_ref[...], preferred_element_type=jnp.float32)
        + fb0_ref[...], 0.0)
    feat = jnp.maximum(
        jnp.dot(feat, fw1_ref[...], preferred_element_type=jnp.float32)
        + fb1_ref[...], 0.0)
    feat = jnp.dot(feat, fw2_ref[...],
                   preferred_element_type=jnp.float32) + fb2_ref[...]

    z = 10.0 * (jnp.dot(feat, dw_ref[...],
                        preferred_element_type=jnp.float32) + db_ref[...])
    raw_d = (jnp.maximum(z, 0.0) + jnp.log1p(jnp.exp(-jnp.abs(z)))) / 10.0
    dens_ref[0] = 1.0 - jnp.exp(-raw_d)

    d_t = dir_t_ref[0]
    nrm = jnp.sqrt(jnp.sum(d_t * d_t, axis=0, keepdims=True))
    rd_t = d_t / jnp.maximum(nrm, 1e-12)
    sins = [jnp.sin(rd_t)]
    coss = [jnp.cos(rd_t)]
    for _ in range(3):
        s_p, c_p = sins[-1], coss[-1]
        sins.append(2.0 * s_p * c_p)
        coss.append(1.0 - 2.0 * s_p * s_p)
    remb_t = jnp.concatenate(sins + coss + [rd_t], axis=0)
    rh = jnp.maximum(
        jnp.dot(feat, rw0a_ref[...], preferred_element_type=jnp.float32)
        + _contract0(remb_t, rw0b_ref[...]) + rb0_ref[...], 0.0)
    rgb = jnp.dot(rh, rw1_ref[...],
                  preferred_element_type=jnp.float32) + rb1_ref[...]
    lane = lax.broadcasted_iota(jnp.int32, rgb.shape, 1)
    srgb = jax.nn.sigmoid(rgb[:, :3]) * (1.0 + 2 * 0.001) - 0.001
    srgb = jnp.concatenate([srgb, rgb[:, 3:]], axis=1)
    rgb_ref[0] = jnp.where(lane < 3, srgb, rgb)


def _decode_call(coordinates, coords_t, dirs_t, dist, s, comb_rows,
                 tex_rows, weights):
    N, M, _ = coordinates.shape
    grid = (N, M // _BM)

    def full(arr):
        nd = arr.ndim
        return pl.BlockSpec(arr.shape, lambda n, mb: (0,) * nd)

    in_specs = [
        pl.BlockSpec((1, _BM, 3), lambda n, mb: (n, mb, 0)),
        pl.BlockSpec((1, 3, _BM), lambda n, mb: (n, 0, mb)),
        pl.BlockSpec((1, 3, _BM), lambda n, mb: (n, 0, mb)),
        pl.BlockSpec((1, _BM, _K), lambda n, mb: (n, mb, 0)),
        pl.BlockSpec((1, 1, _K), lambda n, mb: (n, 0, 0)),
        pl.BlockSpec((1, _BM, _K * 48), lambda n, mb: (n, mb, 0)),
        pl.BlockSpec((1, _BM, 12 * 32), lambda n, mb: (n, mb, 0)),
    ] + [full(w) for w in weights]

    dens, rgb = pl.pallas_call(
        _decode_body,
        grid=grid,
        in_specs=in_specs,
        out_specs=[
            pl.BlockSpec((1, _BM, 1), lambda n, mb: (n, mb, 0)),
            pl.BlockSpec((1, _BM, 32), lambda n, mb: (n, mb, 0)),
        ],
        out_shape=[
            jax.ShapeDtypeStruct((N, M, 1), jnp.float32),
            jax.ShapeDtypeStruct((N, M, 32), jnp.float32),
        ],
        interpret=_INTERPRET,
    )(coordinates, coords_t, dirs_t, dist, s, comb_rows, tex_rows,
      *weights)
    return dens, rgb


def kernel(coordinates, directions, points_position, points_features,
           tex_tplanes, pq_w0, pq_b0, pq_w1, pq_b1, f_w0, f_b0, f_w1, f_b1,
           f_w2, f_b2, d_w, d_b, r_w0, r_b0, r_w1, r_b1):
    N, M, _ = coordinates.shape
    P = points_position.shape[1]

    pts_t = jnp.transpose(points_position, (0, 2, 1))
    comb_tab = jnp.concatenate(
        [points_features, points_position,
         jnp.zeros((N, P, 13), jnp.float32)], axis=-1).reshape(N * P, 48)
    tex_tab = jnp.transpose(
        tex_tplanes, (0, 1, 3, 4, 2)).reshape(N * 3 * _PLANE, 32)

    def fmajor_perm(prefix, nfreq, total):
        perm = list(range(prefix))
        for blk in range(2):
            off = prefix + blk * 3 * nfreq
            perm += [off + a * nfreq + f for f in range(nfreq)
                     for a in range(3)]
        perm += list(range(prefix + 6 * nfreq, total))
        return np.array(perm)

    pq_w0 = pq_w0[fmajor_perm(32, 6, 71)]
    r_w0 = r_w0[fmajor_perm(128, 4, 155)]

    weights = (pq_w0, pq_b0.reshape(1, -1), pq_w1, pq_b1.reshape(1, -1),
               f_w0, f_b0.reshape(1, -1), f_w1, f_b1.reshape(1, -1),
               f_w2, f_b2.reshape(1, -1), d_w, d_b.reshape(1, -1),
               r_w0[:128], r_w0[128:], r_b0.reshape(1, -1),
               r_w1, r_b1.reshape(1, -1))

    coords_t = jnp.transpose(coordinates, (0, 2, 1))
    dirs_t = jnp.transpose(directions, (0, 2, 1))

    n1 = M * _K // (_NW * _CHUNK)
    n2 = M * 12 // (_NW * _CHUNK)
    tex_rows_l = []
    for n in range(N):
        (tex_idx,) = _texidx_call(coordinates[n:n + 1], n)
        tex_rows_l.append(_sc_gather(
            tex_tab, tex_idx.reshape(_NW, n2, _CHUNK)))
    knn = [_knn_call(coordinates[n:n + 1], pts_t[n:n + 1], n)
           for n in range(N)]
    dens_l, rgb_l, dist_l = [], [], []
    for n in range(N):
        dist, idxf, s = knn[n]
        comb_rows = _sc_gather(comb_tab,
                               idxf.reshape(_NW, n1, _CHUNK))
        dens, rgb = _decode_call(
            coordinates[n:n + 1], coords_t[n:n + 1], dirs_t[n:n + 1],
            dist, s, comb_rows.reshape(1, M, _K * 48),
            tex_rows_l[n].reshape(1, M, 12 * 32), weights)
        dens_l.append(dens)
        rgb_l.append(rgb)
        dist_l.append(dist)
    return (jnp.concatenate(dens_l, axis=0), jnp.concatenate(rgb_l, axis=0),
            jnp.concatenate(dist_l, axis=0))


# speedup vs baseline: 1.0542x; 1.0542x over previous
import functools

import numpy as np

import jax
import jax.numpy as jnp
from jax import lax
from jax.experimental import pallas as pl
from jax.experimental.pallas import tpu as pltpu
from jax.experimental.pallas import tpu_sc as plsc

_BM = 512
_K = 8
_RES = 256
_PLANE = _RES * _RES
_CHUNK = 128
_NW = 32

_INTERPRET = False


def _knn_body(coord_ref, pts_ref, dist_ref, idx_ref, s_ref, tex_ref, *,
              P, n0):
    n = n0 + pl.program_id(0)
    mb = pl.program_id(1)
    c = jnp.clip(coord_ref[0], -0.999, 0.999)
    pt = pts_ref[0]
    d2 = jnp.zeros((_BM, P), jnp.float32)
    for a in range(3):
        diff = c[:, a:a + 1] - pt[a:a + 1, :]
        d2 = d2 + diff * diff
    colidx_f = lax.broadcasted_iota(jnp.int32, (_BM, P), 1
                                    ).astype(jnp.float32)
    dcols = []
    icols = []
    for _ in range(_K):
        m = jnp.min(d2, axis=1, keepdims=True)
        am_f = jnp.min(jnp.where(d2 == m, colidx_f, jnp.float32(P)),
                       axis=1, keepdims=True)
        dcols.append(m)
        icols.append(am_f.astype(jnp.int32))
        d2 = jnp.where(colidx_f == am_f, jnp.float32(jnp.inf), d2)
    dist = jnp.concatenate(dcols, axis=1)
    dist_ref[0] = dist
    idx_ref[0] = jnp.concatenate(icols, axis=1) + n * P

    part = jnp.sum(1.0 / dist, axis=0, keepdims=True)

    @pl.when(mb == 0)
    def _():
        s_ref[0] = part

    @pl.when(mb != 0)
    def _():
        s_ref[0] = s_ref[0] + part

    cols = []
    for p, (a, b) in enumerate(((0, 1), (0, 2), (1, 2))):
        u = c[:, a:a + 1]
        v = c[:, b:b + 1]
        x = (u + 1.0) * 0.5 * float(_RES) - 0.5
        y = (v + 1.0) * 0.5 * float(_RES) - 0.5
        x0i = jnp.clip(jnp.floor(x).astype(jnp.int32), 0, _RES - 1)
        y0i = jnp.clip(jnp.floor(y).astype(jnp.int32), 0, _RES - 1)
        x1i = jnp.minimum(x0i + 1, _RES - 1)
        y1i = jnp.minimum(y0i + 1, _RES - 1)
        base = (n * 3 + p) * _PLANE
        cols += [base + y0i * _RES + x0i, base + y0i * _RES + x1i,
                 base + y1i * _RES + x0i, base + y1i * _RES + x1i]
    tex_ref[0] = jnp.concatenate(cols, axis=1)


def _knn_call(coordinates, pts_t, n0):
    N, M, _ = coordinates.shape
    P = pts_t.shape[2]
    grid = (N, M // _BM)
    return pl.pallas_call(
        functools.partial(_knn_body, P=P, n0=n0),
        grid=grid,
        in_specs=[
            pl.BlockSpec((1, _BM, 3), lambda n, mb: (n, mb, 0)),
            pl.BlockSpec((1, 3, P), lambda n, mb: (n, 0, 0)),
        ],
        out_specs=[
            pl.BlockSpec((1, _BM, _K), lambda n, mb: (n, mb, 0)),
            pl.BlockSpec((1, _BM, _K), lambda n, mb: (n, mb, 0)),
            pl.BlockSpec((1, 1, _K), lambda n, mb: (n, 0, 0)),
            pl.BlockSpec((1, _BM, 12), lambda n, mb: (n, mb, 0)),
        ],
        out_shape=[
            jax.ShapeDtypeStruct((N, M, _K), jnp.float32),
            jax.ShapeDtypeStruct((N, M, _K), jnp.int32),
            jax.ShapeDtypeStruct((N, 1, _K), jnp.float32),
            jax.ShapeDtypeStruct((N, M, 12), jnp.int32),
        ],
        interpret=_INTERPRET,
    )(coordinates, pts_t)


def _sc_gather(tab1, tab2, idx1, idx2):
    C1, C2 = idx1.shape[1], idx2.shape[1]
    D1, D2 = tab1.shape[1], tab2.shape[1]
    mesh = plsc.VectorSubcoreMesh(core_axis_name="c", subcore_axis_name="s")

    @functools.partial(
        pl.kernel,
        mesh=mesh,
        out_type=(
            jax.ShapeDtypeStruct((_NW * C1 * _CHUNK, D1), jnp.float32),
            jax.ShapeDtypeStruct((_NW * C2 * _CHUNK, D2), jnp.float32),
        ),
        scratch_types=[
            pltpu.VMEM((C1, _CHUNK), jnp.int32),
            pltpu.VMEM((C2, _CHUNK), jnp.int32),
            pltpu.VMEM((_CHUNK, D1), jnp.float32),
            pltpu.VMEM((_CHUNK, D2), jnp.float32),
            pltpu.SemaphoreType.DMA,
        ],
        compiler_params=pltpu.CompilerParams(use_tc_tiling_on_sc=False),
    )
    def gk(tab1_hbm, tab2_hbm, idx1_hbm, idx2_hbm, out1_hbm, out2_hbm,
           idx1_v, idx2_v, rows1_v, rows2_v, sem):
        wid = lax.axis_index("s") * 2 + lax.axis_index("c")
        pltpu.sync_copy(idx1_hbm.at[wid], idx1_v)
        pltpu.sync_copy(idx2_hbm.at[wid], idx2_v)

        def body1(j, carry):
            pltpu.async_copy(tab1_hbm.at[idx1_v.at[j]], rows1_v, sem).wait()
            pltpu.sync_copy(
                rows1_v,
                out1_hbm.at[pl.ds(wid * (C1 * _CHUNK) + j * _CHUNK,
                                  _CHUNK)])
            return carry

        lax.fori_loop(0, C1, body1, 0)

        def body2(j, carry):
            pltpu.async_copy(tab2_hbm.at[idx2_v.at[j]], rows2_v, sem).wait()
            pltpu.sync_copy(
                rows2_v,
                out2_hbm.at[pl.ds(wid * (C2 * _CHUNK) + j * _CHUNK,
                                  _CHUNK)])
            return carry

        lax.fori_loop(0, C2, body2, 0)

    return gk(tab1, tab2, idx1, idx2)


def _contract0(a, b):
    return lax.dot_general(a, b, (((0,), (0,)), ((), ())),
                           preferred_element_type=jnp.float32)


def _decode_body(coord_ref, coord_t_ref, dir_t_ref, dist_ref, s_ref,
                 comb_ref, tex_ref,
                 pqw0_ref, pqb0_ref, pqw1_ref, pqb1_ref,
                 fw0_ref, fb0_ref, fw1_ref, fb1_ref, fw2_ref, fb2_ref,
                 dw_ref, db_ref, rw0a_ref, rw0b_ref, rb0_ref,
                 rw1_ref, rb1_ref,
                 dens_ref, rgb_ref):
    c = jnp.clip(coord_ref[0], -0.999, 0.999)
    c_t = jnp.clip(coord_t_ref[0], -0.999, 0.999)

    tex_rows = tex_ref[0]
    tex = jnp.zeros((_BM, 32), jnp.float32)
    for p, (a, b) in enumerate(((0, 1), (0, 2), (1, 2))):
        u = c[:, a:a + 1]
        v = c[:, b:b + 1]
        x = (u + 1.0) * 0.5 * float(_RES) - 0.5
        y = (v + 1.0) * 0.5 * float(_RES) - 0.5
        wx = x - jnp.floor(x)
        wy = y - jnp.floor(y)
        w4 = ((1 - wx) * (1 - wy), wx * (1 - wy), (1 - wx) * wy, wx * wy)
        for ci in range(4):
            j = p * 4 + ci
            tex = tex + w4[ci] * tex_rows[:, j * 32:(j + 1) * 32]
    tex = tex / 3.0

    comb = comb_ref[0]
    comb_t = comb.T
    pf_t = jnp.concatenate(
        [comb_t[k * 48:k * 48 + 32, :] for k in range(_K)], axis=1)
    pos_t = jnp.concatenate(
        [comb_t[k * 48 + 32:k * 48 + 35, :] for k in range(_K)], axis=1)
    rel_t = jnp.concatenate([c_t] * _K, axis=1) - pos_t
    sins = [jnp.sin(rel_t)]
    coss = [jnp.cos(rel_t)]
    for _ in range(5):
        s_p, c_p = sins[-1], coss[-1]
        sins.append(2.0 * s_p * c_p)
        coss.append(1.0 - 2.0 * s_p * s_p)
    x_t = jnp.concatenate([pf_t] + sins + coss + [rel_t],
                          axis=0)

    wcols = []
    s = s_ref[0]
    dist = dist_ref[0]
    for k in range(_K):
        wcols.append((1.0 / dist[:, k:k + 1]) / s[0:1, k:k + 1])
    h = jnp.maximum(_contract0(x_t, pqw0_ref[...]) + pqb0_ref[...], 0.0)
    pfo = jnp.dot(h, pqw1_ref[...],
                  preferred_element_type=jnp.float32) + pqb1_ref[...]
    spf = jnp.zeros((_BM, 32), jnp.float32)
    for k in range(_K):
        spf = spf + pfo[k * _BM:(k + 1) * _BM] * wcols[k]

    feat = jnp.concatenate([tex, spf], axis=1)
    feat = jnp.maximum(
        jnp.dot(feat, fw0_ref[...], preferred_element_type=jnp.float32)
        + fb0_ref[...], 0.0)
    feat = jnp.maximum(
        jnp.dot(feat, fw1_ref[...], preferred_element_type=jnp.float32)
        + fb1_ref[...], 0.0)
    feat = jnp.dot(feat, fw2_ref[...],
                   preferred_element_type=jnp.float32) + fb2_ref[...]

    z = 10.0 * (jnp.dot(feat, dw_ref[...],
                        preferred_element_type=jnp.float32) + db_ref[...])
    raw_d = (jnp.maximum(z, 0.0) + jnp.log1p(jnp.exp(-jnp.abs(z)))) / 10.0
    dens_ref[0] = 1.0 - jnp.exp(-raw_d)

    d_t = dir_t_ref[0]
    nrm = jnp.sqrt(jnp.sum(d_t * d_t, axis=0, keepdims=True))
    rd_t = d_t / jnp.maximum(nrm, 1e-12)
    sins = [jnp.sin(rd_t)]
    coss = [jnp.cos(rd_t)]
    for _ in range(3):
        s_p, c_p = sins[-1], coss[-1]
        sins.append(2.0 * s_p * c_p)
        coss.append(1.0 - 2.0 * s_p * s_p)
    remb_t = jnp.concatenate(sins + coss + [rd_t], axis=0)
    rh = jnp.maximum(
        jnp.dot(feat, rw0a_ref[...], preferred_element_type=jnp.float32)
        + _contract0(remb_t, rw0b_ref[...]) + rb0_ref[...], 0.0)
    rgb = jnp.dot(rh, rw1_ref[...],
                  preferred_element_type=jnp.float32) + rb1_ref[...]
    lane = lax.broadcasted_iota(jnp.int32, rgb.shape, 1)
    srgb = jax.nn.sigmoid(rgb[:, :3]) * (1.0 + 2 * 0.001) - 0.001
    srgb = jnp.concatenate([srgb, rgb[:, 3:]], axis=1)
    rgb_ref[0] = jnp.where(lane < 3, srgb, rgb)


def _decode_call(coordinates, coords_t, dirs_t, dist, s, comb_rows,
                 tex_rows, weights):
    N, M, _ = coordinates.shape
    grid = (N, M // _BM)

    def full(arr):
        nd = arr.ndim
        return pl.BlockSpec(arr.shape, lambda n, mb: (0,) * nd)

    in_specs = [
        pl.BlockSpec((1, _BM, 3), lambda n, mb: (n, mb, 0)),
        pl.BlockSpec((1, 3, _BM), lambda n, mb: (n, 0, mb)),
        pl.BlockSpec((1, 3, _BM), lambda n, mb: (n, 0, mb)),
        pl.BlockSpec((1, _BM, _K), lambda n, mb: (n, mb, 0)),
        pl.BlockSpec((1, 1, _K), lambda n, mb: (n, 0, 0)),
        pl.BlockSpec((1, _BM, _K * 48), lambda n, mb: (n, mb, 0)),
        pl.BlockSpec((1, _BM, 12 * 32), lambda n, mb: (n, mb, 0)),
    ] + [full(w) for w in weights]

    dens, rgb = pl.pallas_call(
        _decode_body,
        grid=grid,
        in_specs=in_specs,
        out_specs=[
            pl.BlockSpec((1, _BM, 1), lambda n, mb: (n, mb, 0)),
            pl.BlockSpec((1, _BM, 32), lambda n, mb: (n, mb, 0)),
        ],
        out_shape=[
            jax.ShapeDtypeStruct((N, M, 1), jnp.float32),
            jax.ShapeDtypeStruct((N, M, 32), jnp.float32),
        ],
        interpret=_INTERPRET,
    )(coordinates, coords_t, dirs_t, dist, s, comb_rows, tex_rows,
      *weights)
    return dens, rgb


def kernel(coordinates, directions, points_position, points_features,
           tex_tplanes, pq_w0, pq_b0, pq_w1, pq_b1, f_w0, f_b0, f_w1, f_b1,
           f_w2, f_b2, d_w, d_b, r_w0, r_b0, r_w1, r_b1):
    N, M, _ = coordinates.shape
    P = points_position.shape[1]

    pts_t = jnp.transpose(points_position, (0, 2, 1))
    comb_tab = jnp.concatenate(
        [points_features, points_position,
         jnp.zeros((N, P, 13), jnp.float32)], axis=-1).reshape(N * P, 48)
    tex_tab = jnp.transpose(
        tex_tplanes, (0, 1, 3, 4, 2)).reshape(N * 3 * _PLANE, 32)

    def fmajor_perm(prefix, nfreq, total):
        perm = list(range(prefix))
        for blk in range(2):
            off = prefix + blk * 3 * nfreq
            perm += [off + a * nfreq + f for f in range(nfreq)
                     for a in range(3)]
        perm += list(range(prefix + 6 * nfreq, total))
        return np.array(perm)

    pq_w0 = pq_w0[fmajor_perm(32, 6, 71)]
    r_w0 = r_w0[fmajor_perm(128, 4, 155)]

    weights = (pq_w0, pq_b0.reshape(1, -1), pq_w1, pq_b1.reshape(1, -1),
               f_w0, f_b0.reshape(1, -1), f_w1, f_b1.reshape(1, -1),
               f_w2, f_b2.reshape(1, -1), d_w, d_b.reshape(1, -1),
               r_w0[:128], r_w0[128:], r_b0.reshape(1, -1),
               r_w1, r_b1.reshape(1, -1))

    coords_t = jnp.transpose(coordinates, (0, 2, 1))
    dirs_t = jnp.transpose(directions, (0, 2, 1))

    n1 = M * _K // (_NW * _CHUNK)
    n2 = M * 12 // (_NW * _CHUNK)
    knn = [_knn_call(coordinates[n:n + 1], pts_t[n:n + 1], n)
           for n in range(N)]
    dens_l, rgb_l, dist_l = [], [], []
    for n in range(N):
        dist, idxf, s, tex_idx = knn[n]
        comb_rows, tex_rows = _sc_gather(
            comb_tab, tex_tab,
            idxf.reshape(_NW, n1, _CHUNK),
            tex_idx.reshape(_NW, n2, _CHUNK))
        dens, rgb = _decode_call(
            coordinates[n:n + 1], coords_t[n:n + 1], dirs_t[n:n + 1],
            dist, s, comb_rows.reshape(1, M, _K * 48),
            tex_rows.reshape(1, M, 12 * 32), weights)
        dens_l.append(dens)
        rgb_l.append(rgb)
        dist_l.append(dist)
    return (jnp.concatenate(dens_l, axis=0), jnp.concatenate(rgb_l, axis=0),
            jnp.concatenate(dist_l, axis=0))
